# Initial kernel scaffold; baseline (speedup 1.0000x reference)
import jax
import jax.numpy as jnp
from jax.experimental import pallas as pl


def kernel(x, W):
    raise NotImplementedError("write your pallas kernel here")


# speedup vs baseline: 2.0557x; 2.0557x over previous
import functools

import numpy as np
import jax
import jax.numpy as jnp
from jax import lax
from jax.experimental import pallas as pl
from jax.experimental.pallas import tpu as pltpu
from jax.experimental.pallas import tpu_sc as plsc

EMBED = 128
MAXLEN = 512
LANES = 16


def _make_pe(seq_len: int) -> np.ndarray:
    pe = np.zeros((MAXLEN, EMBED), dtype=np.float32)
    position = np.arange(0, MAXLEN)[:, np.newaxis]
    div_term = np.exp(np.arange(0, EMBED, 2) * -(np.log(10000.0) / EMBED))
    pe[:, 0::2] = np.sin(position * div_term)
    pe[:, 1::2] = np.cos(position * div_term)
    return pe[:seq_len]


@functools.partial(jax.jit, static_argnames=("seq_len",))
def _embed_fixed(xf, W, pe, *, seq_len):
    n_rows = xf.shape[0]
    info = plsc.get_sparse_core_info()
    nc, ns = info.num_cores, info.num_subcores
    nw = nc * ns
    per_w = n_rows // nw
    chunk = seq_len
    n_chunks = per_w // chunk

    mesh = plsc.VectorSubcoreMesh(core_axis_name="c", subcore_axis_name="s")

    @functools.partial(
        pl.kernel,
        mesh=mesh,
        out_type=jax.ShapeDtypeStruct((n_rows, EMBED), jnp.float32),
        scratch_types=[
            pltpu.VMEM((chunk,), jnp.int32),
            pltpu.VMEM((chunk, EMBED), jnp.float32),
            pltpu.VMEM((seq_len, EMBED), jnp.float32),
            pltpu.SemaphoreType.DMA,
        ],
    )
    def body(x_hbm, w_hbm, pe_hbm, out_hbm, idx_v, rows_v, pe_v, sem):
        wid = lax.axis_index("s") * nc + lax.axis_index("c")
        pltpu.sync_copy(pe_hbm, pe_v)

        def chunk_body(c, carry):
            base = wid * per_w + c * chunk
            pltpu.sync_copy(x_hbm.at[pl.ds(base, chunk)], idx_v)
            pltpu.async_copy(w_hbm.at[idx_v], rows_v, sem).wait()

            def row_body(i, carry2):
                for j in range(EMBED // LANES):
                    sl = pl.ds(j * LANES, LANES)
                    rows_v[i, sl] = rows_v[i, sl] + pe_v[i, sl]
                return carry2

            lax.fori_loop(0, chunk, row_body, 0, unroll=2)
            pltpu.sync_copy(rows_v, out_hbm.at[pl.ds(base, chunk)])
            return carry

        lax.fori_loop(0, n_chunks, chunk_body, 0)

    return body(xf, W, pe)


def kernel(x, W):
    b, seq_len = x.shape
    pe = jnp.asarray(_make_pe(seq_len))
    out = _embed_fixed(x.reshape(-1), W, pe, seq_len=seq_len)
    return out.reshape(b, seq_len, EMBED)


# speedup vs baseline: 2.5280x; 1.2297x over previous
import functools

import numpy as np
import jax
import jax.numpy as jnp
from jax import lax
from jax.experimental import pallas as pl
from jax.experimental.pallas import tpu as pltpu
from jax.experimental.pallas import tpu_sc as plsc

EMBED = 128
MAXLEN = 512
LANES = 16


def _make_pe(seq_len: int) -> np.ndarray:
    pe = np.zeros((MAXLEN, EMBED), dtype=np.float32)
    position = np.arange(0, MAXLEN)[:, np.newaxis]
    div_term = np.exp(np.arange(0, EMBED, 2) * -(np.log(10000.0) / EMBED))
    pe[:, 0::2] = np.sin(position * div_term)
    pe[:, 1::2] = np.cos(position * div_term)
    return pe[:seq_len]


@functools.partial(jax.jit, static_argnames=("seq_len",))
def _embed_fixed(xf, W, pe, *, seq_len):
    n_rows = xf.shape[0]
    info = plsc.get_sparse_core_info()
    nc, ns = info.num_cores, info.num_subcores
    nw = nc * ns
    per_w = n_rows // nw
    chunk = seq_len
    n_chunks = per_w // chunk

    mesh = plsc.VectorSubcoreMesh(core_axis_name="c", subcore_axis_name="s")

    @functools.partial(
        pl.kernel,
        mesh=mesh,
        out_type=jax.ShapeDtypeStruct((n_rows, EMBED), jnp.float32),
        scratch_types=[
            pltpu.VMEM((chunk,), jnp.int32),
            pltpu.VMEM((chunk,), jnp.int32),
            pltpu.VMEM((chunk,), jnp.int32),
            pltpu.VMEM((chunk,), jnp.int32),
            pltpu.VMEM((chunk, EMBED), jnp.float32),
            pltpu.VMEM((chunk, EMBED), jnp.float32),
            pltpu.VMEM((chunk, EMBED), jnp.float32),
            pltpu.VMEM((chunk, EMBED), jnp.float32),
            pltpu.VMEM((seq_len, EMBED), jnp.float32),
            pltpu.SemaphoreType.DMA,
            pltpu.SemaphoreType.DMA,
            pltpu.SemaphoreType.DMA,
            pltpu.SemaphoreType.DMA,
            pltpu.SemaphoreType.DMA,
        ],
    )
    def body(x_hbm, w_hbm, pe_hbm, out_hbm,
             ib0, ib1, ib2, ib3, in0, in1, ou0, ou1, pe_v,
             gs0, gs1, os0, os1, isem):
        ibufs = (ib0, ib1, ib2, ib3)
        ins = (in0, in1)
        outs = (ou0, ou1)
        gsems = (gs0, gs1)
        osems = (os0, os1)
        wid = lax.axis_index("s") * nc + lax.axis_index("c")
        base = wid * per_w
        pltpu.sync_copy(pe_hbm, pe_v)

        gd, od, idxd = {}, {}, {}
        for c in (0, 1):
            pltpu.sync_copy(x_hbm.at[pl.ds(base + c * chunk, chunk)],
                            ibufs[c % 4])
            gd[c] = pltpu.async_copy(w_hbm.at[ibufs[c % 4]], ins[c % 2],
                                     gsems[c % 2])

        for c in range(n_chunks):
            b = c % 2
            if c + 2 < n_chunks:
                idxd[c + 2] = pltpu.async_copy(
                    x_hbm.at[pl.ds(base + (c + 2) * chunk, chunk)],
                    ibufs[(c + 2) % 4], isem)
            gd[c].wait()
            if c >= 2:
                od[c - 2].wait()

            def row_body(i, carry, _in=ins[b], _out=outs[b]):
                for j in range(EMBED // LANES):
                    sl = pl.ds(j * LANES, LANES)
                    _out[i, sl] = _in[i, sl] + pe_v[i, sl]
                return carry

            lax.fori_loop(0, chunk, row_body, 0, unroll=2)
            od[c] = pltpu.async_copy(
                outs[b], out_hbm.at[pl.ds(base + c * chunk, chunk)], osems[b])
            if c + 2 < n_chunks:
                idxd[c + 2].wait()
                gd[c + 2] = pltpu.async_copy(w_hbm.at[ibufs[(c + 2) % 4]],
                                             ins[b], gsems[b])

        od[n_chunks - 2].wait()
        od[n_chunks - 1].wait()

    return body(xf, W, pe)


def kernel(x, W):
    b, seq_len = x.shape
    pe = jnp.asarray(_make_pe(seq_len))
    out = _embed_fixed(x.reshape(-1), W, pe, seq_len=seq_len)
    return out.reshape(b, seq_len, EMBED)


# speedup vs baseline: 6.8540x; 2.7113x over previous
import functools

import numpy as np
import jax
import jax.numpy as jnp
from jax import lax
from jax.experimental import pallas as pl
from jax.experimental.pallas import tpu as pltpu
from jax.experimental.pallas import tpu_sc as plsc

EMBED = 128
MAXLEN = 512
LANES = 16


def _make_pe(seq_len: int) -> np.ndarray:
    pe = np.zeros((MAXLEN, EMBED), dtype=np.float32)
    position = np.arange(0, MAXLEN)[:, np.newaxis]
    div_term = np.exp(np.arange(0, EMBED, 2) * -(np.log(10000.0) / EMBED))
    pe[:, 0::2] = np.sin(position * div_term)
    pe[:, 1::2] = np.cos(position * div_term)
    return pe[:seq_len]


@functools.partial(jax.jit, static_argnames=("seq_len",))
def _embed_fixed(xf, W, pe, *, seq_len):
    n_rows = xf.shape[0]
    info = plsc.get_sparse_core_info()
    nc, ns = info.num_cores, info.num_subcores
    nw = nc * ns
    per_w = n_rows // nw
    chunk = seq_len
    n_chunks = per_w // chunk

    mesh = plsc.VectorSubcoreMesh(core_axis_name="c", subcore_axis_name="s")

    @functools.partial(
        pl.kernel,
        mesh=mesh,
        out_type=jax.ShapeDtypeStruct((n_rows, EMBED), jnp.float32),
        scratch_types=[
            pltpu.VMEM((chunk,), jnp.int32),
            pltpu.VMEM((chunk,), jnp.int32),
            pltpu.VMEM((chunk,), jnp.int32),
            pltpu.VMEM((chunk,), jnp.int32),
            pltpu.VMEM((chunk, EMBED), jnp.float32),
            pltpu.VMEM((chunk, EMBED), jnp.float32),
            pltpu.VMEM((chunk, EMBED), jnp.float32),
            pltpu.VMEM((chunk, EMBED), jnp.float32),
            pltpu.VMEM((seq_len, EMBED), jnp.float32),
            pltpu.SemaphoreType.DMA,
            pltpu.SemaphoreType.DMA,
            pltpu.SemaphoreType.DMA,
            pltpu.SemaphoreType.DMA,
            pltpu.SemaphoreType.DMA,
        ],
    )
    def body(x_hbm, w_hbm, pe_hbm, out_hbm,
             ib0, ib1, ib2, ib3, in0, in1, ou0, ou1, pe_v,
             gs0, gs1, os0, os1, isem):
        ibufs = (ib0, ib1, ib2, ib3)
        ins = (in0, in1)
        outs = (ou0, ou1)
        gsems = (gs0, gs1)
        osems = (os0, os1)
        wid = lax.axis_index("s") * nc + lax.axis_index("c")
        base = wid * per_w
        pltpu.sync_copy(pe_hbm, pe_v)

        gd, od, idxd = {}, {}, {}
        for c in (0, 1):
            pltpu.sync_copy(x_hbm.at[pl.ds(base + c * chunk, chunk)],
                            ibufs[c % 4])
            gd[c] = pltpu.async_copy(w_hbm.at[ibufs[c % 4]], ins[c % 2],
                                     gsems[c % 2])

        for c in range(n_chunks):
            b = c % 2
            if c + 2 < n_chunks:
                idxd[c + 2] = pltpu.async_copy(
                    x_hbm.at[pl.ds(base + (c + 2) * chunk, chunk)],
                    ibufs[(c + 2) % 4], isem)
            gd[c].wait()
            if c >= 2:
                od[c - 2].wait()

            @plsc.parallel_loop(0, chunk, unroll=2)
            def row_body(i, _in=ins[b], _out=outs[b]):
                for j in range(EMBED // LANES):
                    sl = pl.ds(j * LANES, LANES)
                    _out[i, sl] = _in[i, sl] + pe_v[i, sl]
            od[c] = pltpu.async_copy(
                outs[b], out_hbm.at[pl.ds(base + c * chunk, chunk)], osems[b])
            if c + 2 < n_chunks:
                idxd[c + 2].wait()
                gd[c + 2] = pltpu.async_copy(w_hbm.at[ibufs[(c + 2) % 4]],
                                             ins[b], gsems[b])

        od[n_chunks - 2].wait()
        od[n_chunks - 1].wait()

    return body(xf, W, pe)


def kernel(x, W):
    b, seq_len = x.shape
    pe = jnp.asarray(_make_pe(seq_len))
    out = _embed_fixed(x.reshape(-1), W, pe, seq_len=seq_len)
    return out.reshape(b, seq_len, EMBED)


# speedup vs baseline: 6.9166x; 1.0091x over previous
import functools

import numpy as np
import jax
import jax.numpy as jnp
from jax import lax
from jax.experimental import pallas as pl
from jax.experimental.pallas import tpu as pltpu
from jax.experimental.pallas import tpu_sc as plsc

EMBED = 128
MAXLEN = 512
LANES = 16


def _make_pe(seq_len: int) -> np.ndarray:
    pe = np.zeros((MAXLEN, EMBED), dtype=np.float32)
    position = np.arange(0, MAXLEN)[:, np.newaxis]
    div_term = np.exp(np.arange(0, EMBED, 2) * -(np.log(10000.0) / EMBED))
    pe[:, 0::2] = np.sin(position * div_term)
    pe[:, 1::2] = np.cos(position * div_term)
    return pe[:seq_len]


@functools.partial(jax.jit, static_argnames=("seq_len",))
def _embed_fixed(x2, W, pe, *, seq_len):
    half = x2.shape[1]
    n_rows = x2.shape[0] * half
    info = plsc.get_sparse_core_info()
    nc, ns = info.num_cores, info.num_subcores
    nw = nc * ns
    per_w = n_rows // nw
    chunk = seq_len
    n_chunks = per_w // chunk

    mesh = plsc.VectorSubcoreMesh(core_axis_name="c", subcore_axis_name="s")

    @functools.partial(
        pl.kernel,
        mesh=mesh,
        out_type=jax.ShapeDtypeStruct((n_rows, EMBED), jnp.float32),
        scratch_types=[
            pltpu.VMEM((8, half), jnp.int32),
            pltpu.VMEM((chunk, EMBED), jnp.float32),
            pltpu.VMEM((chunk, EMBED), jnp.float32),
            pltpu.VMEM((chunk, EMBED), jnp.float32),
            pltpu.VMEM((chunk, EMBED), jnp.float32),
            pltpu.VMEM((seq_len, EMBED), jnp.float32),
            pltpu.SemaphoreType.DMA,
            pltpu.SemaphoreType.DMA,
            pltpu.SemaphoreType.DMA,
            pltpu.SemaphoreType.DMA,
            pltpu.SemaphoreType.DMA,
            pltpu.SemaphoreType.DMA,
            pltpu.SemaphoreType.DMA,
        ],
    )
    def body(x_hbm, w_hbm, pe_hbm, out_hbm,
             ibuf, in0, in1, ou0, ou1, pe_v,
             gs00, gs01, gs10, gs11, os0, os1, isem):
        ins = (in0, in1)
        outs = (ou0, ou1)
        gsems = ((gs00, gs01), (gs10, gs11))
        osems = (os0, os1)
        wid = lax.axis_index("s") * nc + lax.axis_index("c")
        crow0 = wid * n_chunks * 2
        base = wid * per_w
        pltpu.sync_copy(pe_hbm, pe_v)

        gd, od, idxd = {}, {}, {}

        def start_gathers(c):
            b = c % 2
            for h in (0, 1):
                gd[(c, h)] = pltpu.async_copy(
                    w_hbm.at[ibuf.at[(2 * c + h) % 8]],
                    ins[b].at[pl.ds(h * half, half)], gsems[b][h])

        for c in (0, 1):
            for h in (0, 1):
                pltpu.sync_copy(x_hbm.at[crow0 + 2 * c + h],
                                ibuf.at[(2 * c + h) % 8])
            start_gathers(c)

        for c in range(n_chunks):
            b = c % 2
            if c + 2 < n_chunks:
                cc = c + 2
                idxd[cc] = [
                    pltpu.async_copy(x_hbm.at[crow0 + 2 * cc + h],
                                     ibuf.at[(2 * cc + h) % 8], isem)
                    for h in (0, 1)]
            gd[(c, 0)].wait()
            gd[(c, 1)].wait()
            if c >= 2:
                od[c - 2].wait()

            @plsc.parallel_loop(0, chunk, unroll=2)
            def row_body(i, _in=ins[b], _out=outs[b]):
                for j in range(EMBED // LANES):
                    sl = pl.ds(j * LANES, LANES)
                    _out[i, sl] = _in[i, sl] + pe_v[i, sl]

            od[c] = pltpu.async_copy(
                outs[b], out_hbm.at[pl.ds(base + c * chunk, chunk)], osems[b])
            if c + 2 < n_chunks:
                for d in idxd[c + 2]:
                    d.wait()
                start_gathers(c + 2)

        od[n_chunks - 2].wait()
        od[n_chunks - 1].wait()

    return body(x2, W, pe)


def kernel(x, W):
    b, seq_len = x.shape
    pe = jnp.asarray(_make_pe(seq_len))
    x2 = x.reshape(-1, seq_len // 2)
    out = _embed_fixed(x2, W, pe, seq_len=seq_len)
    return out.reshape(b, seq_len, EMBED)
